# initial kernel scaffold (unmeasured)
import jax
import jax.numpy as jnp
from jax import lax
from jax.experimental import pallas as pl
from jax.experimental.pallas import tpu as pltpu


def kernel(
    x,
):
    def body(*refs):
        pass

    out_shape = jax.ShapeDtypeStruct(..., jnp.float32)
    return pl.pallas_call(body, out_shape=out_shape)(...)


# baseline (device time: 375111 ns/iter reference)
import functools

import jax
import jax.numpy as jnp
from jax import lax
from jax.experimental import pallas as pl
from jax.experimental.pallas import tpu as pltpu

N_DEV = 8


def kernel(x):
    m, n = x.shape
    blk = m // N_DEV

    xb = x.astype(jnp.bfloat16)

    def body(x_ref, out_ref, send_buf, rs_recv, send_sems, recv_sems):
        d = lax.axis_index("i")
        left = (d - 1) % N_DEV
        right = (d + 1) % N_DEV

        barrier_sem = pltpu.get_barrier_semaphore()
        for nbr in (left, right):
            pl.semaphore_signal(
                barrier_sem, inc=1,
                device_id=(nbr,), device_id_type=pl.DeviceIdType.MESH,
            )
        pl.semaphore_wait(barrier_sem, 2)

        for h in range(N_DEV - 1):
            blk_idx = (d - h) % N_DEV
            src_rows = pl.ds(blk_idx * blk, blk)
            if h == 0:
                send_buf[...] = x_ref[src_rows]
            else:
                send_buf[...] = rs_recv[h - 1] + x_ref[src_rows]
            rdma = pltpu.make_async_remote_copy(
                src_ref=send_buf,
                dst_ref=rs_recv.at[h],
                send_sem=send_sems.at[h],
                recv_sem=recv_sems.at[h],
                device_id=(right,),
                device_id_type=pl.DeviceIdType.MESH,
            )
            rdma.start()
            rdma.wait()

        own_idx = (d + 1) % N_DEV
        own_rows = pl.ds(own_idx * blk, blk)
        out_ref[own_rows] = rs_recv[N_DEV - 2] + x_ref[own_rows]

        for g in range(N_DEV - 1):
            idx = (d - g + 1) % N_DEV
            rows = pl.ds(idx * blk, blk)
            rdma = pltpu.make_async_remote_copy(
                src_ref=out_ref.at[rows],
                dst_ref=out_ref.at[rows],
                send_sem=send_sems.at[N_DEV - 1 + g],
                recv_sem=recv_sems.at[N_DEV - 1 + g],
                device_id=(right,),
                device_id_type=pl.DeviceIdType.MESH,
            )
            rdma.start()
            rdma.wait()

        @functools.partial(
            pl.run_scoped, second_barrier=pltpu.SemaphoreType.REGULAR
        )
        def _(second_barrier):
            for nbr in (left, right):
                pl.semaphore_signal(
                    second_barrier, inc=1,
                    device_id=(nbr,), device_id_type=pl.DeviceIdType.MESH,
                )
            pl.semaphore_wait(second_barrier, 2)

    return pl.pallas_call(
        body,
        out_shape=jax.ShapeDtypeStruct((m, n), jnp.bfloat16),
        in_specs=[pl.BlockSpec(memory_space=pltpu.VMEM)],
        out_specs=pl.BlockSpec(memory_space=pltpu.VMEM),
        scratch_shapes=[
            pltpu.VMEM((blk, n), jnp.bfloat16),
            pltpu.VMEM((N_DEV - 1, blk, n), jnp.bfloat16),
            pltpu.SemaphoreType.DMA((2 * (N_DEV - 1),)),
            pltpu.SemaphoreType.DMA((2 * (N_DEV - 1),)),
        ],
        compiler_params=pltpu.CompilerParams(collective_id=0),
    )(xb)


# device time: 216511 ns/iter; 1.7325x vs baseline; 1.7325x over previous
import functools

import jax
import jax.numpy as jnp
from jax import lax
from jax.experimental import pallas as pl
from jax.experimental.pallas import tpu as pltpu

N_DEV = 8


def _ring_dev(r):
    return jnp.where(r < 4, r, 11 - r)


def kernel(x):
    m, n = x.shape
    blk = m // N_DEV
    half = n // 2

    xb = x.astype(jnp.bfloat16)

    def body(x_ref, out_ref, send_r, send_l, recv_r, recv_l,
             ssem_r, rsem_r, ssem_l, rsem_l):
        d = lax.axis_index("i")
        r = jnp.where(d < 4, d, 11 - d)
        right = _ring_dev((r + 1) % N_DEV)
        left = _ring_dev((r - 1) % N_DEV)

        cw = pl.ds(0, half)
        ccw = pl.ds(half, half)

        barrier_sem = pltpu.get_barrier_semaphore()
        for nbr in (left, right):
            pl.semaphore_signal(
                barrier_sem, inc=1,
                device_id=(nbr,), device_id_type=pl.DeviceIdType.MESH,
            )
        pl.semaphore_wait(barrier_sem, 2)

        for h in range(N_DEV - 1):
            bi_r = (r - h) % N_DEV
            bi_l = (r + h) % N_DEV
            rows_r = pl.ds(bi_r * blk, blk)
            rows_l = pl.ds(bi_l * blk, blk)
            if h == 0:
                send_r[...] = x_ref[rows_r, cw]
                send_l[...] = x_ref[rows_l, ccw]
            else:
                send_r[...] = recv_r[h - 1] + x_ref[rows_r, cw]
                send_l[...] = recv_l[h - 1] + x_ref[rows_l, ccw]
            rdma_r = pltpu.make_async_remote_copy(
                src_ref=send_r, dst_ref=recv_r.at[h],
                send_sem=ssem_r.at[h], recv_sem=rsem_r.at[h],
                device_id=(right,), device_id_type=pl.DeviceIdType.MESH,
            )
            rdma_l = pltpu.make_async_remote_copy(
                src_ref=send_l, dst_ref=recv_l.at[h],
                send_sem=ssem_l.at[h], recv_sem=rsem_l.at[h],
                device_id=(left,), device_id_type=pl.DeviceIdType.MESH,
            )
            rdma_r.start()
            rdma_l.start()
            rdma_r.wait()
            rdma_l.wait()

        own_r = pl.ds(((r + 1) % N_DEV) * blk, blk)
        own_l = pl.ds(((r - 1) % N_DEV) * blk, blk)
        out_ref[own_r, cw] = recv_r[N_DEV - 2] + x_ref[own_r, cw]
        out_ref[own_l, ccw] = recv_l[N_DEV - 2] + x_ref[own_l, ccw]

        for g in range(N_DEV - 1):
            i_r = (r - g + 1) % N_DEV
            i_l = (r + g - 1) % N_DEV
            rows_r = pl.ds(i_r * blk, blk)
            rows_l = pl.ds(i_l * blk, blk)
            rdma_r = pltpu.make_async_remote_copy(
                src_ref=out_ref.at[rows_r, cw], dst_ref=out_ref.at[rows_r, cw],
                send_sem=ssem_r.at[N_DEV - 1 + g],
                recv_sem=rsem_r.at[N_DEV - 1 + g],
                device_id=(right,), device_id_type=pl.DeviceIdType.MESH,
            )
            rdma_l = pltpu.make_async_remote_copy(
                src_ref=out_ref.at[rows_l, ccw], dst_ref=out_ref.at[rows_l, ccw],
                send_sem=ssem_l.at[N_DEV - 1 + g],
                recv_sem=rsem_l.at[N_DEV - 1 + g],
                device_id=(left,), device_id_type=pl.DeviceIdType.MESH,
            )
            rdma_r.start()
            rdma_l.start()
            rdma_r.wait()
            rdma_l.wait()

        @functools.partial(
            pl.run_scoped, second_barrier=pltpu.SemaphoreType.REGULAR
        )
        def _(second_barrier):
            for nbr in (left, right):
                pl.semaphore_signal(
                    second_barrier, inc=1,
                    device_id=(nbr,), device_id_type=pl.DeviceIdType.MESH,
                )
            pl.semaphore_wait(second_barrier, 2)

    n_sem = 2 * (N_DEV - 1)
    return pl.pallas_call(
        body,
        out_shape=jax.ShapeDtypeStruct((m, n), jnp.bfloat16),
        in_specs=[pl.BlockSpec(memory_space=pltpu.VMEM)],
        out_specs=pl.BlockSpec(memory_space=pltpu.VMEM),
        scratch_shapes=[
            pltpu.VMEM((blk, half), jnp.bfloat16),
            pltpu.VMEM((blk, half), jnp.bfloat16),
            pltpu.VMEM((N_DEV - 1, blk, half), jnp.bfloat16),
            pltpu.VMEM((N_DEV - 1, blk, half), jnp.bfloat16),
            pltpu.SemaphoreType.DMA((n_sem,)),
            pltpu.SemaphoreType.DMA((n_sem,)),
            pltpu.SemaphoreType.DMA((n_sem,)),
            pltpu.SemaphoreType.DMA((n_sem,)),
        ],
        compiler_params=pltpu.CompilerParams(collective_id=0),
    )(xb)
